# Initial kernel scaffold; baseline (speedup 1.0000x reference)
import jax
import jax.numpy as jnp
from jax.experimental import pallas as pl


def kernel(x, adj, W1, b1, W2, b2, beta2):
    raise NotImplementedError("write your pallas kernel here")


# speedup vs baseline: 34.7138x; 34.7138x over previous
import functools

import jax
import jax.numpy as jnp
from jax.experimental import pallas as pl
from jax.experimental.pallas import tpu as pltpu

_F32 = jnp.float32
_NEG = -1e30


def _lin1_body(x_ref, w_ref, b_ref, hT_ref, hnT_ref):
    h = jax.lax.dot_general(w_ref[...], x_ref[...], (((1,), (1,)), ((), ())),
                            preferred_element_type=_F32)
    h = jnp.maximum(h + b_ref[...], 0.0)
    hT_ref[...] = h
    nrm = jnp.sqrt(jnp.sum(h * h, axis=0, keepdims=True))
    hnT_ref[...] = h / jnp.maximum(nrm, 1e-12)


def _prop_body(beta_ref, adj_ref, hT_s_ref, hnT_s_ref, hnT_i_ref,
               oT_ref, onT_ref, acc_ref, m_ref, d_ref, *, bs, bi):
    i = pl.program_id(0)
    s = pl.program_id(1)
    ns = pl.num_programs(1)

    @pl.when(s == 0)
    def _():
        acc_ref[...] = jnp.zeros_like(acc_ref)
        m_ref[...] = jnp.full_like(m_ref, _NEG)
        d_ref[...] = jnp.zeros_like(d_ref)

    beta = beta_ref[0]
    scores = beta * jax.lax.dot_general(
        hnT_s_ref[...], hnT_i_ref[...], (((0,), (0,)), ((), ())),
        preferred_element_type=_F32)
    rows = jax.lax.broadcasted_iota(jnp.int32, (bs, bi), 0) + s * bs
    cols = jax.lax.broadcasted_iota(jnp.int32, (bs, bi), 1) + i * bi
    mult = adj_ref[...] + jnp.where(rows == cols, 1.0, 0.0)
    masked = jnp.where(mult > 0.0, scores, _NEG)
    m_prev = m_ref[...]
    m_new = jnp.maximum(m_prev, jnp.max(masked, axis=0, keepdims=True))
    corr = jnp.exp(m_prev - m_new)
    p = mult * jnp.exp(masked - m_new)
    acc_ref[...] = acc_ref[...] * corr + jax.lax.dot_general(
        hT_s_ref[...], p, (((1,), (0,)), ((), ())),
        preferred_element_type=_F32)
    d_ref[...] = d_ref[...] * corr + jnp.sum(p, axis=0, keepdims=True)
    m_ref[...] = m_new

    @pl.when(s == ns - 1)
    def _():
        o = acc_ref[...] / d_ref[...]
        oT_ref[...] = o
        nrm = jnp.sqrt(jnp.sum(o * o, axis=0, keepdims=True))
        onT_ref[...] = o / jnp.maximum(nrm, 1e-12)


def _lin2_body(hT_ref, w_ref, b_ref, h4T_ref):
    h = jax.lax.dot_general(w_ref[...], hT_ref[...], (((1,), (0,)), ((), ())),
                            preferred_element_type=_F32)
    h4T_ref[...] = jnp.maximum(h + b_ref[...], 0.0)


def _adjmm_body(adj_ref, h4T_ref, out_ref):
    j = pl.program_id(1)

    @pl.when(j == 0)
    def _():
        out_ref[...] = jnp.zeros_like(out_ref)

    out_ref[...] += jax.lax.dot_general(
        adj_ref[...], h4T_ref[...], (((1,), (1,)), ((), ())),
        preferred_element_type=_F32)


def _prop(adj, hT, hnT, beta, *, n, hid, bs, bi, interpret=False):
    ni, ns = n // bi, n // bs
    return pl.pallas_call(
        functools.partial(_prop_body, bs=bs, bi=bi),
        grid=(ni, ns),
        in_specs=[
            pl.BlockSpec(memory_space=pltpu.SMEM),
            pl.BlockSpec((bs, bi), lambda i, s: (s, i)),
            pl.BlockSpec((hid, bs), lambda i, s: (0, s)),
            pl.BlockSpec((hid, bs), lambda i, s: (0, s)),
            pl.BlockSpec((hid, bi), lambda i, s: (0, i)),
        ],
        out_specs=[pl.BlockSpec((hid, bi), lambda i, s: (0, i))] * 2,
        out_shape=[jax.ShapeDtypeStruct((hid, n), _F32)] * 2,
        scratch_shapes=[
            pltpu.VMEM((hid, bi), _F32),
            pltpu.VMEM((1, bi), _F32),
            pltpu.VMEM((1, bi), _F32),
        ],
        interpret=interpret,
    )(beta, adj, hT, hnT, hnT)


def _impl(x, adj, W1, b1, W2, b2, beta2, interpret=False):
    n, in_ch = x.shape
    hid = W1.shape[0]
    bn = min(512, n)
    bs = bi = min(512, n)

    b1c = b1.reshape(hid, 1)
    b2c = b2.reshape(hid, 1)

    hT, hnT = pl.pallas_call(
        _lin1_body,
        grid=(n // bn,),
        in_specs=[
            pl.BlockSpec((bn, in_ch), lambda j: (j, 0)),
            pl.BlockSpec((hid, in_ch), lambda j: (0, 0)),
            pl.BlockSpec((hid, 1), lambda j: (0, 0)),
        ],
        out_specs=[pl.BlockSpec((hid, bn), lambda j: (0, j))] * 2,
        out_shape=[jax.ShapeDtypeStruct((hid, n), _F32)] * 2,
        interpret=interpret,
    )(x, W1, b1c)

    one = jnp.ones((1,), _F32)
    h2T, h2nT = _prop(adj, hT, hnT, one, n=n, hid=hid, bs=bs, bi=bi,
                      interpret=interpret)
    h3T, _ = _prop(adj, h2T, h2nT, beta2.reshape(1).astype(_F32),
                   n=n, hid=hid, bs=bs, bi=bi, interpret=interpret)

    h4T = pl.pallas_call(
        _lin2_body,
        grid=(n // bn,),
        in_specs=[
            pl.BlockSpec((hid, bn), lambda j: (0, j)),
            pl.BlockSpec((hid, hid), lambda j: (0, 0)),
            pl.BlockSpec((hid, 1), lambda j: (0, 0)),
        ],
        out_specs=pl.BlockSpec((hid, bn), lambda j: (0, j)),
        out_shape=jax.ShapeDtypeStruct((hid, n), _F32),
        interpret=interpret,
    )(h3T, W2, b2c)

    bi4 = bj4 = min(512, n)
    out = pl.pallas_call(
        _adjmm_body,
        grid=(n // bi4, n // bj4),
        in_specs=[
            pl.BlockSpec((bi4, bj4), lambda i, j: (i, j)),
            pl.BlockSpec((hid, bj4), lambda i, j: (0, j)),
        ],
        out_specs=pl.BlockSpec((bi4, hid), lambda i, j: (i, 0)),
        out_shape=jax.ShapeDtypeStruct((n, hid), _F32),
        interpret=interpret,
    )(adj, h4T)
    return out


def kernel(x, adj, W1, b1, W2, b2, beta2):
    return _impl(x, adj, W1, b1, W2, b2, beta2)


# speedup vs baseline: 71.2360x; 2.0521x over previous
import functools

import jax
import jax.numpy as jnp
from jax.experimental import pallas as pl
from jax.experimental.pallas import tpu as pltpu

_F32 = jnp.float32
_NEG = -1e30


def _lin1_body(x_ref, w_ref, b_ref, hT_ref, hnT_ref):
    h = jax.lax.dot_general(w_ref[...], x_ref[...], (((1,), (1,)), ((), ())),
                            preferred_element_type=_F32)
    h = jnp.maximum(h + b_ref[...], 0.0)
    hT_ref[...] = h
    nrm = jnp.sqrt(jnp.sum(h * h, axis=0, keepdims=True))
    hnT_ref[...] = h / jnp.maximum(nrm, 1e-12)


def _prop_body(beta_ref, adj_ref, hT_s_ref, hnT_s_ref, hnT_i_ref,
               oT_ref, onT_ref, acc_ref, d_ref, *, bs, bi):
    i = pl.program_id(0)
    s = pl.program_id(1)
    ns = pl.num_programs(1)

    @pl.when(s == 0)
    def _():
        acc_ref[...] = jnp.zeros_like(acc_ref)
        d_ref[...] = jnp.zeros_like(d_ref)

    beta = beta_ref[0]
    scores = beta * jax.lax.dot_general(
        hnT_s_ref[...], hnT_i_ref[...], (((0,), (0,)), ((), ())),
        preferred_element_type=_F32)
    rows = jax.lax.broadcasted_iota(jnp.int32, (bs, bi), 0) + s * bs
    cols = jax.lax.broadcasted_iota(jnp.int32, (bs, bi), 1) + i * bi
    mult = adj_ref[...] + jnp.where(rows == cols, 1.0, 0.0)
    p = mult * jnp.exp(scores)
    acc_ref[...] += jax.lax.dot_general(
        hT_s_ref[...], p, (((1,), (0,)), ((), ())),
        preferred_element_type=_F32)
    d_ref[...] += jnp.sum(p, axis=0, keepdims=True)

    @pl.when(s == ns - 1)
    def _():
        o = acc_ref[...] / d_ref[...]
        oT_ref[...] = o
        nrm = jnp.sqrt(jnp.sum(o * o, axis=0, keepdims=True))
        onT_ref[...] = o / jnp.maximum(nrm, 1e-12)


def _lin2_body(hT_ref, w_ref, b_ref, h4T_ref):
    h = jax.lax.dot_general(w_ref[...], hT_ref[...], (((1,), (0,)), ((), ())),
                            preferred_element_type=_F32)
    h4T_ref[...] = jnp.maximum(h + b_ref[...], 0.0)


def _adjmm_body(adj_ref, h4T_ref, out_ref):
    j = pl.program_id(1)

    @pl.when(j == 0)
    def _():
        out_ref[...] = jnp.zeros_like(out_ref)

    out_ref[...] += jax.lax.dot_general(
        adj_ref[...], h4T_ref[...], (((1,), (1,)), ((), ())),
        preferred_element_type=_F32)


def _prop(adj, hT, hnT, beta, *, n, hid, bs, bi, interpret=False):
    ni, ns = n // bi, n // bs
    return pl.pallas_call(
        functools.partial(_prop_body, bs=bs, bi=bi),
        grid=(ni, ns),
        in_specs=[
            pl.BlockSpec(memory_space=pltpu.SMEM),
            pl.BlockSpec((bs, bi), lambda i, s: (s, i)),
            pl.BlockSpec((hid, bs), lambda i, s: (0, s)),
            pl.BlockSpec((hid, bs), lambda i, s: (0, s)),
            pl.BlockSpec((hid, bi), lambda i, s: (0, i)),
        ],
        out_specs=[pl.BlockSpec((hid, bi), lambda i, s: (0, i))] * 2,
        out_shape=[jax.ShapeDtypeStruct((hid, n), _F32)] * 2,
        scratch_shapes=[
            pltpu.VMEM((hid, bi), _F32),
            pltpu.VMEM((1, bi), _F32),
        ],
        interpret=interpret,
    )(beta, adj, hT, hnT, hnT)


def _impl(x, adj, W1, b1, W2, b2, beta2, interpret=False):
    n, in_ch = x.shape
    hid = W1.shape[0]
    bn = min(512, n)
    bs = bi = min(1024, n)

    b1c = b1.reshape(hid, 1)
    b2c = b2.reshape(hid, 1)

    hT, hnT = pl.pallas_call(
        _lin1_body,
        grid=(n // bn,),
        in_specs=[
            pl.BlockSpec((bn, in_ch), lambda j: (j, 0)),
            pl.BlockSpec((hid, in_ch), lambda j: (0, 0)),
            pl.BlockSpec((hid, 1), lambda j: (0, 0)),
        ],
        out_specs=[pl.BlockSpec((hid, bn), lambda j: (0, j))] * 2,
        out_shape=[jax.ShapeDtypeStruct((hid, n), _F32)] * 2,
        interpret=interpret,
    )(x, W1, b1c)

    one = jnp.ones((1,), _F32)
    h2T, h2nT = _prop(adj, hT, hnT, one, n=n, hid=hid, bs=bs, bi=bi,
                      interpret=interpret)
    h3T, _ = _prop(adj, h2T, h2nT, beta2.reshape(1).astype(_F32),
                   n=n, hid=hid, bs=bs, bi=bi, interpret=interpret)

    h4T = pl.pallas_call(
        _lin2_body,
        grid=(n // bn,),
        in_specs=[
            pl.BlockSpec((hid, bn), lambda j: (0, j)),
            pl.BlockSpec((hid, hid), lambda j: (0, 0)),
            pl.BlockSpec((hid, 1), lambda j: (0, 0)),
        ],
        out_specs=pl.BlockSpec((hid, bn), lambda j: (0, j)),
        out_shape=jax.ShapeDtypeStruct((hid, n), _F32),
        interpret=interpret,
    )(h3T, W2, b2c)

    bi4 = bj4 = min(1024, n)
    out = pl.pallas_call(
        _adjmm_body,
        grid=(n // bi4, n // bj4),
        in_specs=[
            pl.BlockSpec((bi4, bj4), lambda i, j: (i, j)),
            pl.BlockSpec((hid, bj4), lambda i, j: (0, j)),
        ],
        out_specs=pl.BlockSpec((bi4, hid), lambda i, j: (i, 0)),
        out_shape=jax.ShapeDtypeStruct((n, hid), _F32),
        interpret=interpret,
    )(adj, h4T)
    return out


def kernel(x, adj, W1, b1, W2, b2, beta2):
    return _impl(x, adj, W1, b1, W2, b2, beta2)


# speedup vs baseline: 73.5592x; 1.0326x over previous
import functools

import jax
import jax.numpy as jnp
from jax.experimental import pallas as pl
from jax.experimental.pallas import tpu as pltpu

_F32 = jnp.float32


def _lin1_body(x_ref, w_ref, b_ref, hT_ref, hnT_ref):
    h = jax.lax.dot_general(w_ref[...], x_ref[...], (((1,), (1,)), ((), ())),
                            preferred_element_type=_F32)
    h = jnp.maximum(h + b_ref[...], 0.0)
    hT_ref[...] = h
    nrm = jnp.sqrt(jnp.sum(h * h, axis=0, keepdims=True))
    hnT_ref[...] = h / jnp.maximum(nrm, 1e-12)


def _prop_body(beta_ref, adj_ref, hT_s_ref, hnT_s_ref, hnT_i_ref,
               oT_ref, onT_ref, *rest, bs, bi, emit_i8):
    if emit_i8:
        adj_i8_ref, acc_ref, d_ref = rest
    else:
        acc_ref, d_ref = rest
    i = pl.program_id(0)
    s = pl.program_id(1)
    ns = pl.num_programs(1)

    @pl.when(s == 0)
    def _():
        acc_ref[...] = jnp.zeros_like(acc_ref)
        d_ref[...] = jnp.zeros_like(d_ref)

    beta = beta_ref[0]
    scores = beta * jax.lax.dot_general(
        hnT_s_ref[...], hnT_i_ref[...], (((0,), (0,)), ((), ())),
        preferred_element_type=_F32)
    a = adj_ref[...]
    if emit_i8:
        adj_i8_ref[...] = a.astype(jnp.int8)
        af = a
    else:
        af = a.astype(_F32)
    rows = jax.lax.broadcasted_iota(jnp.int32, (bs, bi), 0) + s * bs
    cols = jax.lax.broadcasted_iota(jnp.int32, (bs, bi), 1) + i * bi
    mult = af + jnp.where(rows == cols, 1.0, 0.0)
    p = mult * jnp.exp(scores)
    acc_ref[...] += jax.lax.dot_general(
        hT_s_ref[...], p, (((1,), (0,)), ((), ())),
        preferred_element_type=_F32)
    d_ref[...] += jnp.sum(p, axis=0, keepdims=True)

    @pl.when(s == ns - 1)
    def _():
        o = acc_ref[...] / d_ref[...]
        oT_ref[...] = o
        nrm = jnp.sqrt(jnp.sum(o * o, axis=0, keepdims=True))
        onT_ref[...] = o / jnp.maximum(nrm, 1e-12)


def _lin2_body(hT_ref, w_ref, b_ref, h4T_ref):
    h = jax.lax.dot_general(w_ref[...], hT_ref[...], (((1,), (0,)), ((), ())),
                            preferred_element_type=_F32)
    h4T_ref[...] = jnp.maximum(h + b_ref[...], 0.0)


def _adjmm_body(adj_ref, h4T_ref, out_ref):
    j = pl.program_id(1)

    @pl.when(j == 0)
    def _():
        out_ref[...] = jnp.zeros_like(out_ref)

    out_ref[...] += jax.lax.dot_general(
        adj_ref[...].astype(_F32), h4T_ref[...], (((1,), (1,)), ((), ())),
        preferred_element_type=_F32)


def _prop(adj, hT, hnT, beta, *, n, hid, bs, bi, emit_i8, interpret=False):
    ni, ns = n // bi, n // bs
    out_shape = [jax.ShapeDtypeStruct((hid, n), _F32)] * 2
    out_specs = [pl.BlockSpec((hid, bi), lambda i, s: (0, i))] * 2
    if emit_i8:
        out_shape = out_shape + [jax.ShapeDtypeStruct((n, n), jnp.int8)]
        out_specs = out_specs + [pl.BlockSpec((bs, bi), lambda i, s: (s, i))]
    return pl.pallas_call(
        functools.partial(_prop_body, bs=bs, bi=bi, emit_i8=emit_i8),
        grid=(ni, ns),
        in_specs=[
            pl.BlockSpec(memory_space=pltpu.SMEM),
            pl.BlockSpec((bs, bi), lambda i, s: (s, i)),
            pl.BlockSpec((hid, bs), lambda i, s: (0, s)),
            pl.BlockSpec((hid, bs), lambda i, s: (0, s)),
            pl.BlockSpec((hid, bi), lambda i, s: (0, i)),
        ],
        out_specs=out_specs,
        out_shape=out_shape,
        scratch_shapes=[
            pltpu.VMEM((hid, bi), _F32),
            pltpu.VMEM((1, bi), _F32),
        ],
        interpret=interpret,
    )(beta, adj, hT, hnT, hnT)


def _impl(x, adj, W1, b1, W2, b2, beta2, interpret=False):
    n, in_ch = x.shape
    hid = W1.shape[0]
    bn = min(512, n)
    bs = bi = min(1024, n)

    b1c = b1.reshape(hid, 1)
    b2c = b2.reshape(hid, 1)

    hT, hnT = pl.pallas_call(
        _lin1_body,
        grid=(n // bn,),
        in_specs=[
            pl.BlockSpec((bn, in_ch), lambda j: (j, 0)),
            pl.BlockSpec((hid, in_ch), lambda j: (0, 0)),
            pl.BlockSpec((hid, 1), lambda j: (0, 0)),
        ],
        out_specs=[pl.BlockSpec((hid, bn), lambda j: (0, j))] * 2,
        out_shape=[jax.ShapeDtypeStruct((hid, n), _F32)] * 2,
        interpret=interpret,
    )(x, W1, b1c)

    one = jnp.ones((1,), _F32)
    h2T, h2nT, adj_i8 = _prop(adj, hT, hnT, one, n=n, hid=hid, bs=bs, bi=bi,
                              emit_i8=True, interpret=interpret)
    h3T, _ = _prop(adj_i8, h2T, h2nT, beta2.reshape(1).astype(_F32),
                   n=n, hid=hid, bs=bs, bi=bi, emit_i8=False,
                   interpret=interpret)

    h4T = pl.pallas_call(
        _lin2_body,
        grid=(n // bn,),
        in_specs=[
            pl.BlockSpec((hid, bn), lambda j: (0, j)),
            pl.BlockSpec((hid, hid), lambda j: (0, 0)),
            pl.BlockSpec((hid, 1), lambda j: (0, 0)),
        ],
        out_specs=pl.BlockSpec((hid, bn), lambda j: (0, j)),
        out_shape=jax.ShapeDtypeStruct((hid, n), _F32),
        interpret=interpret,
    )(h3T, W2, b2c)

    bi4 = bj4 = min(1024, n)
    out = pl.pallas_call(
        _adjmm_body,
        grid=(n // bi4, n // bj4),
        in_specs=[
            pl.BlockSpec((bi4, bj4), lambda i, j: (i, j)),
            pl.BlockSpec((hid, bj4), lambda i, j: (0, j)),
        ],
        out_specs=pl.BlockSpec((bi4, hid), lambda i, j: (i, 0)),
        out_shape=jax.ShapeDtypeStruct((n, hid), _F32),
        interpret=interpret,
    )(adj_i8, h4T)
    return out


def kernel(x, adj, W1, b1, W2, b2, beta2):
    return _impl(x, adj, W1, b1, W2, b2, beta2)


# speedup vs baseline: 79.3900x; 1.0793x over previous
import functools

import jax
import jax.numpy as jnp
from jax.experimental import pallas as pl
from jax.experimental.pallas import tpu as pltpu

_F32 = jnp.float32
_PAD = 8


def _lin1_body(x_ref, w_ref, b_ref, hTe_ref, hnT_ref, *, hid):
    h = jax.lax.dot_general(w_ref[...], x_ref[...], (((1,), (1,)), ((), ())),
                            preferred_element_type=_F32)
    h = jnp.maximum(h + b_ref[...], 0.0)
    hTe_ref[0:hid, :] = h
    hTe_ref[hid:, :] = jnp.ones_like(hTe_ref[hid:, :])
    nrm = jnp.sqrt(jnp.sum(h * h, axis=0, keepdims=True))
    hnT_ref[...] = h / jnp.maximum(nrm, 1e-12)


def _diag_update(acc_ref, hTe, e, bs, bi):
    r = jax.lax.broadcasted_iota(jnp.int32, (bs, bi), 0)
    c = jax.lax.broadcasted_iota(jnp.int32, (bs, bi), 1)
    pd = jnp.where(r == c, e, 0.0)
    acc_ref[...] += jax.lax.dot_general(
        hTe, pd, (((1,), (0,)), ((), ())), preferred_element_type=_F32)


def _prop1_body(beta2_ref, adj_ref, hTe_ref, hnT_ref, hniT_ref,
                oTe_ref, onT_ref, obnT_ref, adj8_ref, acc_ref,
                *, bs, bi, hid):
    i = pl.program_id(0)
    s = pl.program_id(1)
    ns = pl.num_programs(1)

    @pl.when(s == 0)
    def _():
        acc_ref[...] = jnp.zeros_like(acc_ref)

    e = jnp.exp(jax.lax.dot_general(
        hnT_ref[...], hniT_ref[...], (((0,), (0,)), ((), ())),
        preferred_element_type=_F32))
    a = adj_ref[...]
    adj8_ref[...] = a.astype(jnp.int8)
    p = a * e
    acc_ref[...] += jax.lax.dot_general(
        hTe_ref[...], p, (((1,), (0,)), ((), ())),
        preferred_element_type=_F32)

    @pl.when(i == s)
    def _():
        _diag_update(acc_ref, hTe_ref[...], e, bs, bi)

    @pl.when(s == ns - 1)
    def _():
        o = acc_ref[0:hid, :] / acc_ref[hid:hid + 1, :]
        oTe_ref[0:hid, :] = o
        oTe_ref[hid:, :] = jnp.ones_like(oTe_ref[hid:, :])
        nrm = jnp.sqrt(jnp.sum(o * o, axis=0, keepdims=True))
        on = o / jnp.maximum(nrm, 1e-12)
        onT_ref[...] = on
        obnT_ref[...] = beta2_ref[0] * on


def _prop2_body(adj8_ref, hTe_ref, hnT_ref, hniT_ref, oTe_ref, acc_ref,
                *, bs, bi, hid):
    i = pl.program_id(0)
    s = pl.program_id(1)
    ns = pl.num_programs(1)

    @pl.when(s == 0)
    def _():
        acc_ref[...] = jnp.zeros_like(acc_ref)

    e = jnp.exp(jax.lax.dot_general(
        hnT_ref[...], hniT_ref[...], (((0,), (0,)), ((), ())),
        preferred_element_type=_F32))
    p = jnp.where(adj8_ref[...] != 0, e, 0.0)
    acc_ref[...] += jax.lax.dot_general(
        hTe_ref[...], p, (((1,), (0,)), ((), ())),
        preferred_element_type=_F32)

    @pl.when(i == s)
    def _():
        _diag_update(acc_ref, hTe_ref[...], e, bs, bi)

    @pl.when(s == ns - 1)
    def _():
        o = acc_ref[0:hid, :] / acc_ref[hid:hid + 1, :]
        oTe_ref[0:hid, :] = o
        oTe_ref[hid:, :] = jnp.ones_like(oTe_ref[hid:, :])


def _lin2_body(hTe_ref, w_ref, b_ref, h4T_ref, *, hid):
    h = jax.lax.dot_general(
        w_ref[...], hTe_ref[0:hid, :], (((1,), (0,)), ((), ())),
        preferred_element_type=_F32)
    h4T_ref[...] = jnp.maximum(h + b_ref[...], 0.0)


def _adjmm_body(adj_ref, h4T_ref, out_ref):
    j = pl.program_id(1)

    @pl.when(j == 0)
    def _():
        out_ref[...] = jnp.zeros_like(out_ref)

    out_ref[...] += jax.lax.dot_general(
        adj_ref[...].astype(_F32), h4T_ref[...], (((1,), (1,)), ((), ())),
        preferred_element_type=_F32)


def _impl(x, adj, W1, b1, W2, b2, beta2, interpret=False):
    n, in_ch = x.shape
    hid = W1.shape[0]
    he = hid + _PAD
    bn = min(512, n)
    bs = bi = min(1024, n)
    ni, ns = n // bi, n // bs

    b1c = b1.reshape(hid, 1)
    b2c = b2.reshape(hid, 1)

    hTe, hnT = pl.pallas_call(
        functools.partial(_lin1_body, hid=hid),
        grid=(n // bn,),
        in_specs=[
            pl.BlockSpec((bn, in_ch), lambda j: (j, 0)),
            pl.BlockSpec((hid, in_ch), lambda j: (0, 0)),
            pl.BlockSpec((hid, 1), lambda j: (0, 0)),
        ],
        out_specs=[pl.BlockSpec((he, bn), lambda j: (0, j)),
                   pl.BlockSpec((hid, bn), lambda j: (0, j))],
        out_shape=[jax.ShapeDtypeStruct((he, n), _F32),
                   jax.ShapeDtypeStruct((hid, n), _F32)],
        interpret=interpret,
    )(x, W1, b1c)

    h2Te, h2nT, h2bnT, adj_i8 = pl.pallas_call(
        functools.partial(_prop1_body, bs=bs, bi=bi, hid=hid),
        grid=(ni, ns),
        in_specs=[
            pl.BlockSpec(memory_space=pltpu.SMEM),
            pl.BlockSpec((bs, bi), lambda i, s: (s, i)),
            pl.BlockSpec((he, bs), lambda i, s: (0, s)),
            pl.BlockSpec((hid, bs), lambda i, s: (0, s)),
            pl.BlockSpec((hid, bi), lambda i, s: (0, i)),
        ],
        out_specs=[
            pl.BlockSpec((he, bi), lambda i, s: (0, i)),
            pl.BlockSpec((hid, bi), lambda i, s: (0, i)),
            pl.BlockSpec((hid, bi), lambda i, s: (0, i)),
            pl.BlockSpec((bs, bi), lambda i, s: (s, i)),
        ],
        out_shape=[
            jax.ShapeDtypeStruct((he, n), _F32),
            jax.ShapeDtypeStruct((hid, n), _F32),
            jax.ShapeDtypeStruct((hid, n), _F32),
            jax.ShapeDtypeStruct((n, n), jnp.int8),
        ],
        scratch_shapes=[pltpu.VMEM((he, bi), _F32)],
        interpret=interpret,
    )(beta2.reshape(1).astype(_F32), adj, hTe, hnT, hnT)

    h3Te = pl.pallas_call(
        functools.partial(_prop2_body, bs=bs, bi=bi, hid=hid),
        grid=(ni, ns),
        in_specs=[
            pl.BlockSpec((bs, bi), lambda i, s: (s, i)),
            pl.BlockSpec((he, bs), lambda i, s: (0, s)),
            pl.BlockSpec((hid, bs), lambda i, s: (0, s)),
            pl.BlockSpec((hid, bi), lambda i, s: (0, i)),
        ],
        out_specs=pl.BlockSpec((he, bi), lambda i, s: (0, i)),
        out_shape=jax.ShapeDtypeStruct((he, n), _F32),
        scratch_shapes=[pltpu.VMEM((he, bi), _F32)],
        interpret=interpret,
    )(adj_i8, h2Te, h2nT, h2bnT)

    h4T = pl.pallas_call(
        functools.partial(_lin2_body, hid=hid),
        grid=(n // bn,),
        in_specs=[
            pl.BlockSpec((he, bn), lambda j: (0, j)),
            pl.BlockSpec((hid, hid), lambda j: (0, 0)),
            pl.BlockSpec((hid, 1), lambda j: (0, 0)),
        ],
        out_specs=pl.BlockSpec((hid, bn), lambda j: (0, j)),
        out_shape=jax.ShapeDtypeStruct((hid, n), _F32),
        interpret=interpret,
    )(h3Te, W2, b2c)

    bi4 = bj4 = min(1024, n)
    out = pl.pallas_call(
        _adjmm_body,
        grid=(n // bi4, n // bj4),
        in_specs=[
            pl.BlockSpec((bi4, bj4), lambda i, j: (i, j)),
            pl.BlockSpec((hid, bj4), lambda i, j: (0, j)),
        ],
        out_specs=pl.BlockSpec((bi4, hid), lambda i, j: (i, 0)),
        out_shape=jax.ShapeDtypeStruct((n, hid), _F32),
        interpret=interpret,
    )(adj_i8, h4T)
    return out


def kernel(x, adj, W1, b1, W2, b2, beta2):
    return _impl(x, adj, W1, b1, W2, b2, beta2)


# speedup vs baseline: 83.4508x; 1.0512x over previous
import functools

import jax
import jax.numpy as jnp
from jax.experimental import pallas as pl
from jax.experimental.pallas import tpu as pltpu

_F32 = jnp.float32
_BF16 = jnp.bfloat16
_PAD = 8


def _lin1_body(x_ref, w_ref, b_ref, hTe_ref, hnT_ref, *, hid):
    h = jax.lax.dot_general(w_ref[...], x_ref[...], (((1,), (1,)), ((), ())),
                            preferred_element_type=_F32)
    h = jnp.maximum(h + b_ref[...], 0.0)
    hTe_ref[0:hid, :] = h
    hTe_ref[hid:, :] = jnp.ones_like(hTe_ref[hid:, :])
    nrm = jnp.sqrt(jnp.sum(h * h, axis=0, keepdims=True))
    hnT_ref[...] = (h / jnp.maximum(nrm, 1e-12)).astype(_BF16)


def _diag_update(acc_ref, hTe, e, bs, bi):
    r = jax.lax.broadcasted_iota(jnp.int32, (bs, bi), 0)
    c = jax.lax.broadcasted_iota(jnp.int32, (bs, bi), 1)
    pd = jnp.where(r == c, e, 0.0)
    acc_ref[...] += jax.lax.dot_general(
        hTe, pd, (((1,), (0,)), ((), ())), preferred_element_type=_F32)


def _prop1_body(beta2_ref, adj_ref, hTe_ref, hnT_ref, hniT_ref,
                oTe_ref, onT_ref, obnT_ref, adj8_ref, acc_ref,
                *, bs, bi, hid):
    i = pl.program_id(0)
    s = pl.program_id(1)
    ns = pl.num_programs(1)

    @pl.when(s == 0)
    def _():
        acc_ref[...] = jnp.zeros_like(acc_ref)

    e = jnp.exp(jax.lax.dot_general(
        hnT_ref[...], hniT_ref[...], (((0,), (0,)), ((), ())),
        preferred_element_type=_F32))
    a = adj_ref[...]
    adj8_ref[...] = a.astype(jnp.int8)
    p = a * e
    acc_ref[...] += jax.lax.dot_general(
        hTe_ref[...], p, (((1,), (0,)), ((), ())),
        preferred_element_type=_F32)

    @pl.when(i == s)
    def _():
        _diag_update(acc_ref, hTe_ref[...], e, bs, bi)

    @pl.when(s == ns - 1)
    def _():
        o = acc_ref[0:hid, :] / acc_ref[hid:hid + 1, :]
        oTe_ref[0:hid, :] = o
        oTe_ref[hid:, :] = jnp.ones_like(oTe_ref[hid:, :])
        nrm = jnp.sqrt(jnp.sum(o * o, axis=0, keepdims=True))
        on = o / jnp.maximum(nrm, 1e-12)
        onT_ref[...] = on.astype(_BF16)
        obnT_ref[...] = (beta2_ref[0] * on).astype(_BF16)


def _prop2_body(adj8_ref, hTe_ref, hnT_ref, hniT_ref, w2_ref, b2_ref,
                h4T_ref, acc_ref, *, bs, bi, hid):
    i = pl.program_id(0)
    s = pl.program_id(1)
    ns = pl.num_programs(1)

    @pl.when(s == 0)
    def _():
        acc_ref[...] = jnp.zeros_like(acc_ref)

    e = jnp.exp(jax.lax.dot_general(
        hnT_ref[...], hniT_ref[...], (((0,), (0,)), ((), ())),
        preferred_element_type=_F32))
    p = jnp.where(adj8_ref[...] != 0, e, 0.0)
    acc_ref[...] += jax.lax.dot_general(
        hTe_ref[...], p, (((1,), (0,)), ((), ())),
        preferred_element_type=_F32)

    @pl.when(i == s)
    def _():
        _diag_update(acc_ref, hTe_ref[...], e, bs, bi)

    @pl.when(s == ns - 1)
    def _():
        o = acc_ref[0:hid, :] / acc_ref[hid:hid + 1, :]
        h4 = jax.lax.dot_general(
            w2_ref[...], o, (((1,), (0,)), ((), ())),
            preferred_element_type=_F32)
        h4T_ref[...] = jnp.maximum(h4 + b2_ref[...], 0.0).astype(_BF16)


def _adjmm_body(adj_ref, h4T_ref, out_ref):
    j = pl.program_id(1)

    @pl.when(j == 0)
    def _():
        out_ref[...] = jnp.zeros_like(out_ref)

    out_ref[...] += jax.lax.dot_general(
        adj_ref[...].astype(_BF16), h4T_ref[...], (((1,), (1,)), ((), ())),
        preferred_element_type=_F32)


def _impl(x, adj, W1, b1, W2, b2, beta2, interpret=False):
    n, in_ch = x.shape
    hid = W1.shape[0]
    he = hid + _PAD
    bn = min(512, n)
    bs = bi = min(1024, n)
    ni, ns = n // bi, n // bs

    b1c = b1.reshape(hid, 1)
    b2c = b2.reshape(hid, 1)

    hTe, hnT = pl.pallas_call(
        functools.partial(_lin1_body, hid=hid),
        grid=(n // bn,),
        in_specs=[
            pl.BlockSpec((bn, in_ch), lambda j: (j, 0)),
            pl.BlockSpec((hid, in_ch), lambda j: (0, 0)),
            pl.BlockSpec((hid, 1), lambda j: (0, 0)),
        ],
        out_specs=[pl.BlockSpec((he, bn), lambda j: (0, j)),
                   pl.BlockSpec((hid, bn), lambda j: (0, j))],
        out_shape=[jax.ShapeDtypeStruct((he, n), _F32),
                   jax.ShapeDtypeStruct((hid, n), _BF16)],
        interpret=interpret,
    )(x, W1, b1c)

    h2Te, h2nT, h2bnT, adj_i8 = pl.pallas_call(
        functools.partial(_prop1_body, bs=bs, bi=bi, hid=hid),
        grid=(ni, ns),
        in_specs=[
            pl.BlockSpec(memory_space=pltpu.SMEM),
            pl.BlockSpec((bs, bi), lambda i, s: (s, i)),
            pl.BlockSpec((he, bs), lambda i, s: (0, s)),
            pl.BlockSpec((hid, bs), lambda i, s: (0, s)),
            pl.BlockSpec((hid, bi), lambda i, s: (0, i)),
        ],
        out_specs=[
            pl.BlockSpec((he, bi), lambda i, s: (0, i)),
            pl.BlockSpec((hid, bi), lambda i, s: (0, i)),
            pl.BlockSpec((hid, bi), lambda i, s: (0, i)),
            pl.BlockSpec((bs, bi), lambda i, s: (s, i)),
        ],
        out_shape=[
            jax.ShapeDtypeStruct((he, n), _F32),
            jax.ShapeDtypeStruct((hid, n), _BF16),
            jax.ShapeDtypeStruct((hid, n), _BF16),
            jax.ShapeDtypeStruct((n, n), jnp.int8),
        ],
        scratch_shapes=[pltpu.VMEM((he, bi), _F32)],
        interpret=interpret,
    )(beta2.reshape(1).astype(_F32), adj, hTe, hnT, hnT)

    h4T = pl.pallas_call(
        functools.partial(_prop2_body, bs=bs, bi=bi, hid=hid),
        grid=(ni, ns),
        in_specs=[
            pl.BlockSpec((bs, bi), lambda i, s: (s, i)),
            pl.BlockSpec((he, bs), lambda i, s: (0, s)),
            pl.BlockSpec((hid, bs), lambda i, s: (0, s)),
            pl.BlockSpec((hid, bi), lambda i, s: (0, i)),
            pl.BlockSpec((hid, hid), lambda i, s: (0, 0)),
            pl.BlockSpec((hid, 1), lambda i, s: (0, 0)),
        ],
        out_specs=pl.BlockSpec((hid, bi), lambda i, s: (0, i)),
        out_shape=jax.ShapeDtypeStruct((hid, n), _BF16),
        scratch_shapes=[pltpu.VMEM((he, bi), _F32)],
        interpret=interpret,
    )(adj_i8, h2Te, h2nT, h2bnT, W2, b2c)

    bi4 = bj4 = min(1024, n)
    out = pl.pallas_call(
        _adjmm_body,
        grid=(n // bi4, n // bj4),
        in_specs=[
            pl.BlockSpec((bi4, bj4), lambda i, j: (i, j)),
            pl.BlockSpec((hid, bj4), lambda i, j: (0, j)),
        ],
        out_specs=pl.BlockSpec((bi4, hid), lambda i, j: (i, 0)),
        out_shape=jax.ShapeDtypeStruct((n, hid), _F32),
        interpret=interpret,
    )(adj_i8, h4T)
    return out


def kernel(x, adj, W1, b1, W2, b2, beta2):
    return _impl(x, adj, W1, b1, W2, b2, beta2)


# speedup vs baseline: 85.2770x; 1.0219x over previous
import functools

import jax
import jax.numpy as jnp
from jax.experimental import pallas as pl
from jax.experimental.pallas import tpu as pltpu

_F32 = jnp.float32
_BF16 = jnp.bfloat16
_PAD = 8


def _lin1_body(x_ref, w_ref, b_ref, hTe_ref, hnT_ref, *, hid):
    h = jax.lax.dot_general(w_ref[...], x_ref[...], (((1,), (1,)), ((), ())),
                            preferred_element_type=_F32)
    h = jnp.maximum(h + b_ref[...], 0.0)
    hTe_ref[0:hid, :] = h.astype(_BF16)
    hTe_ref[hid:, :] = jnp.ones_like(hTe_ref[hid:, :])
    nrm = jnp.sqrt(jnp.sum(h * h, axis=0, keepdims=True))
    hnT_ref[...] = (h / jnp.maximum(nrm, 1e-12)).astype(_BF16)


def _diag_update(acc_ref, hTe, e, bs, bi):
    r = jax.lax.broadcasted_iota(jnp.int32, (bs, bi), 0)
    c = jax.lax.broadcasted_iota(jnp.int32, (bs, bi), 1)
    pd = jnp.where(r == c, e, 0.0).astype(_BF16)
    acc_ref[...] += jax.lax.dot_general(
        hTe, pd, (((1,), (0,)), ((), ())), preferred_element_type=_F32)


def _prop1_body(beta2_ref, adj_ref, hTe_ref, hnT_ref, hniT_ref,
                oTe_ref, onT_ref, obnT_ref, adj8_ref, acc_ref,
                *, bs, bi, hid):
    i = pl.program_id(0)
    s = pl.program_id(1)
    ns = pl.num_programs(1)

    @pl.when(s == 0)
    def _():
        acc_ref[...] = jnp.zeros_like(acc_ref)

    e = jnp.exp(jax.lax.dot_general(
        hnT_ref[...], hniT_ref[...], (((0,), (0,)), ((), ())),
        preferred_element_type=_F32))
    a = adj_ref[...]
    adj8_ref[...] = a.astype(jnp.int8)
    p = (a * e).astype(_BF16)
    acc_ref[...] += jax.lax.dot_general(
        hTe_ref[...], p, (((1,), (0,)), ((), ())),
        preferred_element_type=_F32)

    @pl.when(i == s)
    def _():
        _diag_update(acc_ref, hTe_ref[...], e, bs, bi)

    @pl.when(s == ns - 1)
    def _():
        o = acc_ref[0:hid, :] / acc_ref[hid:hid + 1, :]
        oTe_ref[0:hid, :] = o.astype(_BF16)
        oTe_ref[hid:, :] = jnp.ones_like(oTe_ref[hid:, :])
        nrm = jnp.sqrt(jnp.sum(o * o, axis=0, keepdims=True))
        on = o / jnp.maximum(nrm, 1e-12)
        onT_ref[...] = on.astype(_BF16)
        obnT_ref[...] = (beta2_ref[0] * on).astype(_BF16)


def _prop2_body(adj8_ref, hTe_ref, hnT_ref, hniT_ref, w2_ref, b2_ref,
                h4T_ref, acc_ref, *, bs, bi, hid):
    i = pl.program_id(0)
    s = pl.program_id(1)
    ns = pl.num_programs(1)

    @pl.when(s == 0)
    def _():
        acc_ref[...] = jnp.zeros_like(acc_ref)

    e = jnp.exp(jax.lax.dot_general(
        hnT_ref[...], hniT_ref[...], (((0,), (0,)), ((), ())),
        preferred_element_type=_F32))
    p = jnp.where(adj8_ref[...] != 0, e, 0.0).astype(_BF16)
    acc_ref[...] += jax.lax.dot_general(
        hTe_ref[...], p, (((1,), (0,)), ((), ())),
        preferred_element_type=_F32)

    @pl.when(i == s)
    def _():
        _diag_update(acc_ref, hTe_ref[...], e, bs, bi)

    @pl.when(s == ns - 1)
    def _():
        o = acc_ref[0:hid, :] / acc_ref[hid:hid + 1, :]
        h4 = jax.lax.dot_general(
            w2_ref[...], o, (((1,), (0,)), ((), ())),
            preferred_element_type=_F32)
        h4T_ref[...] = jnp.maximum(h4 + b2_ref[...], 0.0).astype(_BF16)


def _adjmm_body(adj_ref, h4T_ref, out_ref):
    j = pl.program_id(1)

    @pl.when(j == 0)
    def _():
        out_ref[...] = jnp.zeros_like(out_ref)

    out_ref[...] += jax.lax.dot_general(
        adj_ref[...].astype(_BF16), h4T_ref[...], (((1,), (1,)), ((), ())),
        preferred_element_type=_F32)


def _impl(x, adj, W1, b1, W2, b2, beta2, interpret=False):
    n, in_ch = x.shape
    hid = W1.shape[0]
    he = hid + _PAD
    bn = min(512, n)
    bs = bi = min(1024, n)
    ni, ns = n // bi, n // bs

    b1c = b1.reshape(hid, 1)
    b2c = b2.reshape(hid, 1)

    hTe, hnT = pl.pallas_call(
        functools.partial(_lin1_body, hid=hid),
        grid=(n // bn,),
        in_specs=[
            pl.BlockSpec((bn, in_ch), lambda j: (j, 0)),
            pl.BlockSpec((hid, in_ch), lambda j: (0, 0)),
            pl.BlockSpec((hid, 1), lambda j: (0, 0)),
        ],
        out_specs=[pl.BlockSpec((he, bn), lambda j: (0, j)),
                   pl.BlockSpec((hid, bn), lambda j: (0, j))],
        out_shape=[jax.ShapeDtypeStruct((he, n), _BF16),
                   jax.ShapeDtypeStruct((hid, n), _BF16)],
        interpret=interpret,
    )(x, W1, b1c)

    h2Te, h2nT, h2bnT, adj_i8 = pl.pallas_call(
        functools.partial(_prop1_body, bs=bs, bi=bi, hid=hid),
        grid=(ni, ns),
        in_specs=[
            pl.BlockSpec(memory_space=pltpu.SMEM),
            pl.BlockSpec((bs, bi), lambda i, s: (s, i)),
            pl.BlockSpec((he, bs), lambda i, s: (0, s)),
            pl.BlockSpec((hid, bs), lambda i, s: (0, s)),
            pl.BlockSpec((hid, bi), lambda i, s: (0, i)),
        ],
        out_specs=[
            pl.BlockSpec((he, bi), lambda i, s: (0, i)),
            pl.BlockSpec((hid, bi), lambda i, s: (0, i)),
            pl.BlockSpec((hid, bi), lambda i, s: (0, i)),
            pl.BlockSpec((bs, bi), lambda i, s: (s, i)),
        ],
        out_shape=[
            jax.ShapeDtypeStruct((he, n), _BF16),
            jax.ShapeDtypeStruct((hid, n), _BF16),
            jax.ShapeDtypeStruct((hid, n), _BF16),
            jax.ShapeDtypeStruct((n, n), jnp.int8),
        ],
        scratch_shapes=[pltpu.VMEM((he, bi), _F32)],
        interpret=interpret,
    )(beta2.reshape(1).astype(_F32), adj, hTe, hnT, hnT)

    h4T = pl.pallas_call(
        functools.partial(_prop2_body, bs=bs, bi=bi, hid=hid),
        grid=(ni, ns),
        in_specs=[
            pl.BlockSpec((bs, bi), lambda i, s: (s, i)),
            pl.BlockSpec((he, bs), lambda i, s: (0, s)),
            pl.BlockSpec((hid, bs), lambda i, s: (0, s)),
            pl.BlockSpec((hid, bi), lambda i, s: (0, i)),
            pl.BlockSpec((hid, hid), lambda i, s: (0, 0)),
            pl.BlockSpec((hid, 1), lambda i, s: (0, 0)),
        ],
        out_specs=pl.BlockSpec((hid, bi), lambda i, s: (0, i)),
        out_shape=jax.ShapeDtypeStruct((hid, n), _BF16),
        scratch_shapes=[pltpu.VMEM((he, bi), _F32)],
        interpret=interpret,
    )(adj_i8, h2Te, h2nT, h2bnT, W2, b2c)

    bi4 = bj4 = min(1024, n)
    out = pl.pallas_call(
        _adjmm_body,
        grid=(n // bi4, n // bj4),
        in_specs=[
            pl.BlockSpec((bi4, bj4), lambda i, j: (i, j)),
            pl.BlockSpec((hid, bj4), lambda i, j: (0, j)),
        ],
        out_specs=pl.BlockSpec((bi4, hid), lambda i, j: (i, 0)),
        out_shape=jax.ShapeDtypeStruct((n, hid), _F32),
        interpret=interpret,
    )(adj_i8, h4T)
    return out


def kernel(x, adj, W1, b1, W2, b2, beta2):
    return _impl(x, adj, W1, b1, W2, b2, beta2)
